# Initial kernel scaffold; baseline (speedup 1.0000x reference)
import jax
import jax.numpy as jnp
from jax.experimental import pallas as pl


def kernel(waveform, W1, b1, W2, b2, codebooks):
    raise NotImplementedError("write your pallas kernel here")


# speedup vs baseline: 36.8476x; 36.8476x over previous
import functools
import math

import jax
import jax.numpy as jnp
import numpy as np
from jax.experimental import pallas as pl

SR = 24000
N_FFT = 1024
HOP = 320
N_MELS = 128
VOCAB = 1024
NCB = 4
DM = 512
NFRAMES = 751
YROWS = 754
HP = jax.lax.Precision.HIGHEST
DP = jax.lax.Precision.DEFAULT


def _mel_fb_np():
    n_freqs = N_FFT // 2 + 1
    all_freqs = np.linspace(0.0, SR / 2.0, n_freqs)

    def hz_to_mel(f):
        return 2595.0 * np.log10(1.0 + f / 700.0)

    def mel_to_hz(m):
        return 700.0 * (10.0 ** (m / 2595.0) - 1.0)

    m_pts = np.linspace(hz_to_mel(0.0), hz_to_mel(SR / 2.0), N_MELS + 2)
    f_pts = mel_to_hz(m_pts)
    f_diff = f_pts[1:] - f_pts[:-1]
    slopes = f_pts[None, :] - all_freqs[:, None]
    down = -slopes[:, :-2] / f_diff[:-1]
    up = slopes[:, 2:] / f_diff[1:]
    return np.maximum(0.0, np.minimum(down, up))


NFREQ = N_FFT // 2 + 1
FPAD = 640


@functools.lru_cache(maxsize=1)
def _dft_constants():
    n = np.arange(N_FFT)
    win = 0.5 - 0.5 * np.cos(2.0 * np.pi * n / N_FFT)
    f = np.arange(NFREQ)
    ang = 2.0 * np.pi * n[:, None] * f[None, :] / N_FFT
    gc = np.zeros((N_FFT, FPAD))
    gs = np.zeros((N_FFT, FPAD))
    gc[:, :NFREQ] = win[:, None] * np.cos(ang)
    gs[:, :NFREQ] = win[:, None] * np.sin(ang)
    g = np.concatenate([gc, gs], axis=1)
    w = np.zeros((FPAD, N_MELS))
    w[:NFREQ] = _mel_fb_np()
    return np.asarray(g, np.float32), np.asarray(w, np.float32)


def _tokenizer_kernel(y_ref, g_ref, w_ref, a1_ref, b1_ref, a2_ref, b2_ref,
                      cbt_ref, out_ref):
    y = y_ref[0]
    frames = jnp.concatenate(
        [y[0:NFRAMES], y[1 : NFRAMES + 1], y[2 : NFRAMES + 2],
         y[3 : NFRAMES + 3, :N_FFT - 3 * HOP]],
        axis=1,
    )
    u = jnp.dot(frames, g_ref[...], precision=HP,
                preferred_element_type=jnp.float32)
    power = u[:, :FPAD] ** 2 + u[:, FPAD:] ** 2
    mel = jnp.dot(power, w_ref[...], precision=DP,
                  preferred_element_type=jnp.float32)
    mel = jnp.log(jnp.clip(mel, 1e-5, None))

    zc = jnp.zeros((1, N_MELS), jnp.float32)
    melp = jnp.concatenate([zc, mel, zc], axis=0)
    a1 = a1_ref[...]
    h = (jnp.dot(melp[0:NFRAMES], a1[0:128], precision=DP,
                 preferred_element_type=jnp.float32)
         + jnp.dot(melp[1 : NFRAMES + 1], a1[128:256], precision=DP,
                   preferred_element_type=jnp.float32)
         + jnp.dot(melp[2 : NFRAMES + 2], a1[256:384], precision=DP,
                   preferred_element_type=jnp.float32)
         + b1_ref[...])
    h = 0.5 * h * (1.0 + jax.lax.erf(h * (1.0 / math.sqrt(2.0))))

    zh = jnp.zeros((1, 256), jnp.float32)
    hp = jnp.concatenate([zh, h, zh], axis=0)
    a2 = a2_ref[...]
    f = (jnp.dot(hp[0:NFRAMES], a2[0:256], precision=DP,
                 preferred_element_type=jnp.float32)
         + jnp.dot(hp[1 : NFRAMES + 1], a2[256:512], precision=DP,
                   preferred_element_type=jnp.float32)
         + jnp.dot(hp[2 : NFRAMES + 2], a2[512:768], precision=DP,
                   preferred_element_type=jnp.float32)
         + b2_ref[...])

    d = DM // NCB
    idx = jax.lax.broadcasted_iota(jnp.int32, (NFRAMES, VOCAB), 1)
    toks = []
    for i in range(NCB):
        cbt = cbt_ref[i]
        cn = jnp.sum(cbt * cbt, axis=0, keepdims=True)
        s = jnp.dot(f[:, i * d : (i + 1) * d], cbt, precision=DP,
                    preferred_element_type=jnp.float32)
        scores = cn - 2.0 * s
        m = jnp.min(scores, axis=-1, keepdims=True)
        toks.append(jnp.min(jnp.where(scores == m, idx, VOCAB), axis=-1)
                    .astype(jnp.int32))
    out_ref[0] = jnp.stack(toks, axis=0)


def kernel(waveform, W1, b1, W2, b2, codebooks):
    B = waveform.shape[0]
    g_np, w_np = _dft_constants()
    g = jnp.asarray(g_np)
    w = jnp.asarray(w_np)

    pad = N_FFT // 2
    xp = jnp.pad(waveform, ((0, 0), (pad, pad)), mode='reflect')
    xp = jnp.pad(xp, ((0, 0), (0, YROWS * HOP - xp.shape[1])))
    y = xp.reshape(B, YROWS, HOP)

    a1 = jnp.concatenate([W1[:, :, k].T for k in range(3)], axis=0)
    a2 = jnp.concatenate([W2[:, :, k].T for k in range(3)], axis=0)
    b1r = b1.reshape(1, -1)
    b2r = b2.reshape(1, -1)
    cbt = jnp.transpose(codebooks, (0, 2, 1))

    const = lambda shape: pl.BlockSpec(shape, lambda b: (0,) * len(shape))
    out = pl.pallas_call(
        _tokenizer_kernel,
        grid=(B,),
        in_specs=[
            pl.BlockSpec((1, YROWS, HOP), lambda b: (b, 0, 0)),
            const((N_FFT, 2 * FPAD)),
            const((FPAD, N_MELS)),
            const((384, 256)),
            const((1, 256)),
            const((768, 512)),
            const((1, 512)),
            const((NCB, DM // NCB, VOCAB)),
        ],
        out_specs=pl.BlockSpec((1, NCB, NFRAMES), lambda b: (b, 0, 0)),
        out_shape=jax.ShapeDtypeStruct((B, NCB, NFRAMES), jnp.int32),
    )(y, g, w, a1, b1r, a2, b2r, cbt)
    return out
